# Initial kernel scaffold; baseline (speedup 1.0000x reference)
import jax
import jax.numpy as jnp
from jax.experimental import pallas as pl


def kernel(x, w1, w2, g1, b1, g2, b2):
    raise NotImplementedError("write your pallas kernel here")


# speedup vs baseline: 1.1192x; 1.1192x over previous
import functools

import jax
import jax.numpy as jnp
from jax import lax
from jax.experimental import pallas as pl
from jax.experimental.pallas import tpu as pltpu

BN_EPS = 1e-5
VMEM_LIMIT_BYTES = 48 * 1024 * 1024


def _conv_body(v, w_ref, h, w):
    c = v.shape[-1]
    vm1 = jnp.concatenate([v[:, -1:], v[:, :-1]], axis=1)
    vp1 = jnp.concatenate([v[:, 1:], v[:, :1]], axis=1)
    xc = jnp.concatenate([vm1, v, vp1], axis=2)
    xcp = jnp.concatenate([xc[-1:], xc, xc[:1]], axis=0)
    xb = xcp.reshape((h + 2) * w, 3 * c)
    rows = h * w
    acc = jnp.dot(xb[0:rows], w_ref[0], preferred_element_type=jnp.float32)
    acc = acc + jnp.dot(xb[w:w + rows], w_ref[1],
                        preferred_element_type=jnp.float32)
    acc = acc + jnp.dot(xb[2 * w:2 * w + rows], w_ref[2],
                        preferred_element_type=jnp.float32)
    return acc


def _accum_stats(stats_ref, acc):
    ts = jnp.sum(acc, axis=0, keepdims=True)
    tq = jnp.sum(acc * acc, axis=0, keepdims=True)
    tot = jnp.concatenate([ts, tq], axis=0)

    @pl.when(pl.program_id(1) == 0)
    def _init():
        stats_ref[...] = jnp.zeros_like(stats_ref)

    stats_ref[...] += tot


def _conv1_kernel(x_ref, w_ref, y_ref, stats_ref, *, h, w):
    v = x_ref[...].astype(jnp.bfloat16).reshape(h, w, x_ref.shape[-1])
    acc = _conv_body(v, w_ref, h, w)
    y_ref[...] = acc.astype(jnp.bfloat16)
    _accum_stats(stats_ref, acc)


def _conv2_kernel(y1_ref, w_ref, ss_ref, y2_ref, stats_ref, *, h, w):
    a = jnp.maximum(y1_ref[...].astype(jnp.float32) * ss_ref[0] + ss_ref[1], 0.0)
    v = a.astype(jnp.bfloat16).reshape(h, w, a.shape[-1])
    acc = _conv_body(v, w_ref, h, w)
    y2_ref[...] = acc.astype(jnp.bfloat16)
    _accum_stats(stats_ref, acc)


def _bn_out_kernel(y2_ref, ss_ref, o_ref):
    a = jnp.maximum(y2_ref[...].astype(jnp.float32) * ss_ref[0] + ss_ref[1], 0.0)
    o_ref[...] = a.T


def _fold_bn(stats, gamma, beta, count):
    mean = stats[0] / count
    var = jnp.maximum(stats[1] / count - mean * mean, 0.0)
    inv = lax.rsqrt(var + BN_EPS)
    scale = gamma.astype(jnp.float32) * inv
    shift = beta.astype(jnp.float32) - mean * scale
    return jnp.stack([scale, shift], axis=0)


def _wt(weight):
    co, ci = weight.shape[0], weight.shape[1]
    return jnp.transpose(weight, (2, 3, 1, 0)).reshape(3, 3 * ci, co).astype(
        jnp.bfloat16)


def _conv_stats(x2d, wt, ss, kern_fn, h, w, n, co, ncores):
    per_core = n // ncores
    rows = h * w
    cin = x2d.shape[-1]
    kern = functools.partial(kern_fn, h=h, w=w)
    in_specs = [
        pl.BlockSpec((rows, cin), lambda c, i: (c * per_core + i, 0)),
        pl.BlockSpec(wt.shape, lambda c, i: (0, 0, 0)),
    ]
    args = [x2d, wt]
    if ss is not None:
        in_specs.append(pl.BlockSpec((2, co), lambda c, i: (0, 0)))
        args.append(ss)
    y, stats = pl.pallas_call(
        kern,
        out_shape=(jax.ShapeDtypeStruct((n * rows, co), jnp.bfloat16),
                   jax.ShapeDtypeStruct((2, ncores * co), jnp.float32)),
        grid=(ncores, per_core),
        in_specs=in_specs,
        out_specs=(
            pl.BlockSpec((rows, co), lambda c, i: (c * per_core + i, 0)),
            pl.BlockSpec((2, co), lambda c, i: (0, c)),
        ),
        compiler_params=pltpu.CompilerParams(
            dimension_semantics=("parallel", "arbitrary"),
            vmem_limit_bytes=VMEM_LIMIT_BYTES),
    )(*args)
    return y, stats.reshape(2, ncores, co).sum(axis=1)


def kernel(x, w1, w2, g1, b1, g2, b2):
    n, cin, hh, ww = x.shape
    h, w = hh // 2, ww // 2
    cmid, cout = w1.shape[0], w2.shape[0]
    rows = h * w
    cnt = jnp.float32(n * rows)
    ncores = 2 if n % 2 == 0 else 1
    per_core = n // ncores

    p = jnp.maximum(
        jnp.maximum(x[:, :, 0::2, 0::2], x[:, :, 0::2, 1::2]),
        jnp.maximum(x[:, :, 1::2, 0::2], x[:, :, 1::2, 1::2]))
    x2d = jnp.transpose(p, (0, 2, 3, 1)).reshape(n * rows, cin)

    y1, stats1 = _conv_stats(x2d, _wt(w1), None, _conv1_kernel,
                             h, w, n, cmid, ncores)
    ss1 = _fold_bn(stats1, g1, b1, cnt)

    y2, stats2 = _conv_stats(y1, _wt(w2), ss1, _conv2_kernel,
                             h, w, n, cout, ncores)
    ss2 = _fold_bn(stats2, g2, b2, cnt)

    out = pl.pallas_call(
        _bn_out_kernel,
        out_shape=jax.ShapeDtypeStruct((n * cout, rows), jnp.float32),
        grid=(ncores, per_core),
        in_specs=[
            pl.BlockSpec((rows, cout), lambda c, i: (c * per_core + i, 0)),
            pl.BlockSpec((2, cout), lambda c, i: (0, 0)),
        ],
        out_specs=pl.BlockSpec((cout, rows), lambda c, i: (c * per_core + i, 0)),
        compiler_params=pltpu.CompilerParams(
            dimension_semantics=("parallel", "arbitrary"),
            vmem_limit_bytes=VMEM_LIMIT_BYTES),
    )(y2, ss2)
    return out.reshape(n, cout, h, w)


# speedup vs baseline: 5.9567x; 5.3224x over previous
import functools

import jax
import jax.numpy as jnp
from jax import lax
from jax.experimental import pallas as pl
from jax.experimental.pallas import tpu as pltpu

BN_EPS = 1e-5
VMEM_LIMIT_BYTES = 48 * 1024 * 1024


def _conv_body(v, w_ref, h, w):
    c = v.shape[-1]
    vm1 = jnp.concatenate([v[:, -1:], v[:, :-1]], axis=1)
    vp1 = jnp.concatenate([v[:, 1:], v[:, :1]], axis=1)
    xc = jnp.concatenate([vm1, v, vp1], axis=2)
    xcp = jnp.concatenate([xc[-1:], xc, xc[:1]], axis=0)
    xb = xcp.reshape((h + 2) * w, 3 * c)
    rows = h * w
    acc = jnp.dot(xb[0:rows], w_ref[0], preferred_element_type=jnp.float32)
    acc = acc + jnp.dot(xb[w:w + rows], w_ref[1],
                        preferred_element_type=jnp.float32)
    acc = acc + jnp.dot(xb[2 * w:2 * w + rows], w_ref[2],
                        preferred_element_type=jnp.float32)
    return acc


def _accum_stats(stats_ref, acc):
    ts = jnp.sum(acc, axis=0, keepdims=True)
    tq = jnp.sum(acc * acc, axis=0, keepdims=True)
    tot = jnp.concatenate([ts, tq], axis=0)

    @pl.when(pl.program_id(1) == 0)
    def _init():
        stats_ref[...] = jnp.zeros_like(stats_ref)

    stats_ref[...] += tot


def _conv1_kernel(x_ref, w_ref, y_ref, stats_ref, *, h, w):
    c = x_ref.shape[0]
    v = x_ref[...].astype(jnp.bfloat16)
    hm = jnp.max(v.reshape(c, h, 2, 2 * w), axis=2)
    t = jnp.transpose(hm, (1, 2, 0))
    vt = jnp.max(t.reshape(h, w, 2, c), axis=2)
    acc = _conv_body(vt, w_ref, h, w)
    y_ref[...] = acc.astype(jnp.bfloat16)
    _accum_stats(stats_ref, acc)


def _conv2_kernel(y1_ref, w_ref, ss_ref, y2_ref, stats_ref, *, h, w):
    a = jnp.maximum(y1_ref[...].astype(jnp.float32) * ss_ref[0] + ss_ref[1], 0.0)
    v = a.astype(jnp.bfloat16).reshape(h, w, a.shape[-1])
    acc = _conv_body(v, w_ref, h, w)
    y2_ref[...] = acc.astype(jnp.bfloat16)
    _accum_stats(stats_ref, acc)


def _bn_out_kernel(y2_ref, ss_ref, o_ref):
    a = jnp.maximum(y2_ref[...].astype(jnp.float32) * ss_ref[0] + ss_ref[1], 0.0)
    o_ref[...] = a.T


def _fold_bn(stats, gamma, beta, count):
    mean = stats[0] / count
    var = jnp.maximum(stats[1] / count - mean * mean, 0.0)
    inv = lax.rsqrt(var + BN_EPS)
    scale = gamma.astype(jnp.float32) * inv
    shift = beta.astype(jnp.float32) - mean * scale
    return jnp.stack([scale, shift], axis=0)


def _wt(weight):
    co, ci = weight.shape[0], weight.shape[1]
    return jnp.transpose(weight, (2, 3, 1, 0)).reshape(3, 3 * ci, co).astype(
        jnp.bfloat16)


def _conv_stats(x, x_spec, wt, ss, kern_fn, h, w, n, co, ncores):
    per_core = n // ncores
    rows = h * w
    kern = functools.partial(kern_fn, h=h, w=w)
    in_specs = [
        x_spec,
        pl.BlockSpec(wt.shape, lambda c, i: (0, 0, 0)),
    ]
    args = [x, wt]
    if ss is not None:
        in_specs.append(pl.BlockSpec((2, co), lambda c, i: (0, 0)))
        args.append(ss)
    y, stats = pl.pallas_call(
        kern,
        out_shape=(jax.ShapeDtypeStruct((n * rows, co), jnp.bfloat16),
                   jax.ShapeDtypeStruct((2, ncores * co), jnp.float32)),
        grid=(ncores, per_core),
        in_specs=in_specs,
        out_specs=(
            pl.BlockSpec((rows, co), lambda c, i: (c * per_core + i, 0)),
            pl.BlockSpec((2, co), lambda c, i: (0, c)),
        ),
        compiler_params=pltpu.CompilerParams(
            dimension_semantics=("parallel", "arbitrary"),
            vmem_limit_bytes=VMEM_LIMIT_BYTES),
    )(*args)
    return y, stats.reshape(2, ncores, co).sum(axis=1)


def kernel(x, w1, w2, g1, b1, g2, b2):
    n, cin, hh, ww = x.shape
    h, w = hh // 2, ww // 2
    cmid, cout = w1.shape[0], w2.shape[0]
    rows = h * w
    cnt = jnp.float32(n * rows)
    ncores = 2 if n % 2 == 0 else 1
    per_core = n // ncores

    x3 = x.reshape(n * cin, hh, ww)
    xa_spec = pl.BlockSpec((cin, hh, ww), lambda c, i: (c * per_core + i, 0, 0))
    y1, stats1 = _conv_stats(x3, xa_spec, _wt(w1), None, _conv1_kernel,
                             h, w, n, cmid, ncores)
    ss1 = _fold_bn(stats1, g1, b1, cnt)

    xb_spec = pl.BlockSpec((rows, cmid), lambda c, i: (c * per_core + i, 0))
    y2, stats2 = _conv_stats(y1, xb_spec, _wt(w2), ss1, _conv2_kernel,
                             h, w, n, cout, ncores)
    ss2 = _fold_bn(stats2, g2, b2, cnt)

    out = pl.pallas_call(
        _bn_out_kernel,
        out_shape=jax.ShapeDtypeStruct((n * cout, rows), jnp.float32),
        grid=(ncores, per_core),
        in_specs=[
            pl.BlockSpec((rows, cout), lambda c, i: (c * per_core + i, 0)),
            pl.BlockSpec((2, cout), lambda c, i: (0, 0)),
        ],
        out_specs=pl.BlockSpec((cout, rows), lambda c, i: (c * per_core + i, 0)),
        compiler_params=pltpu.CompilerParams(
            dimension_semantics=("parallel", "arbitrary"),
            vmem_limit_bytes=VMEM_LIMIT_BYTES),
    )(y2, ss2)
    return out.reshape(n, cout, h, w)


# speedup vs baseline: 9.5406x; 1.6017x over previous
import functools

import jax
import jax.numpy as jnp
from jax import lax
from jax.experimental import pallas as pl
from jax.experimental.pallas import tpu as pltpu

BN_EPS = 1e-5
VMEM_LIMIT_BYTES = 48 * 1024 * 1024


def _conv_body(v, w_ref, h, w):
    c = v.shape[-1]
    vm1 = jnp.concatenate([v[:, -1:], v[:, :-1]], axis=1)
    vp1 = jnp.concatenate([v[:, 1:], v[:, :1]], axis=1)
    xc = jnp.concatenate([vm1, v, vp1], axis=2)
    xcp = jnp.concatenate([xc[-1:], xc, xc[:1]], axis=0)
    xb = xcp.reshape((h + 2) * w, 3 * c)
    rows = h * w
    acc = jnp.dot(xb[0:rows], w_ref[0], preferred_element_type=jnp.float32)
    acc = acc + jnp.dot(xb[w:w + rows], w_ref[1],
                        preferred_element_type=jnp.float32)
    acc = acc + jnp.dot(xb[2 * w:2 * w + rows], w_ref[2],
                        preferred_element_type=jnp.float32)
    return acc


def _accum_stats(stats_ref, acc):
    ts = jnp.sum(acc, axis=0, keepdims=True)
    tq = jnp.sum(acc * acc, axis=0, keepdims=True)
    tot = jnp.concatenate([ts, tq], axis=0)

    @pl.when(pl.program_id(1) == 0)
    def _init():
        stats_ref[...] = jnp.zeros_like(stats_ref)

    stats_ref[...] += tot


def _conv1_kernel(xe_ref, xo_ref, w_ref, y_ref, stats_ref, *, h, w):
    c = xe_ref.shape[0]
    hm = jnp.maximum(xe_ref[...], xo_ref[...])
    ms = jnp.maximum(hm, jnp.concatenate([hm[:, :, 1:], hm[:, :, :1]], axis=2))
    mb = ms.astype(jnp.bfloat16).reshape(c * h, 2 * w)
    sel = (lax.broadcasted_iota(jnp.int32, (2 * w, w), 0)
           == 2 * lax.broadcasted_iota(jnp.int32, (2 * w, w), 1)
           ).astype(jnp.bfloat16)
    wp = jnp.dot(mb, sel, preferred_element_type=jnp.float32)
    vt = jnp.transpose(wp.astype(jnp.bfloat16).reshape(c, h, w), (1, 2, 0))
    acc = _conv_body(vt, w_ref, h, w)
    y_ref[...] = acc.astype(jnp.bfloat16)
    _accum_stats(stats_ref, acc)


def _conv2_kernel(y1_ref, w_ref, ss_ref, y2_ref, stats_ref, *, h, w):
    a = jnp.maximum(y1_ref[...].astype(jnp.float32) * ss_ref[0] + ss_ref[1], 0.0)
    v = a.astype(jnp.bfloat16).reshape(h, w, a.shape[-1])
    acc = _conv_body(v, w_ref, h, w)
    y2_ref[...] = acc.astype(jnp.bfloat16)
    _accum_stats(stats_ref, acc)


def _bn_out_kernel(y2_ref, ss_ref, o_ref):
    a = jnp.maximum(y2_ref[...].astype(jnp.float32) * ss_ref[0] + ss_ref[1], 0.0)
    o_ref[...] = a.T


def _fold_bn(stats, gamma, beta, count):
    mean = stats[0] / count
    var = jnp.maximum(stats[1] / count - mean * mean, 0.0)
    inv = lax.rsqrt(var + BN_EPS)
    scale = gamma.astype(jnp.float32) * inv
    shift = beta.astype(jnp.float32) - mean * scale
    return jnp.stack([scale, shift], axis=0)


def _wt(weight):
    co, ci = weight.shape[0], weight.shape[1]
    return jnp.transpose(weight, (2, 3, 1, 0)).reshape(3, 3 * ci, co).astype(
        jnp.bfloat16)


def _conv_stats(xs, x_specs, wt, ss, kern_fn, h, w, n, co, ncores):
    per_core = n // ncores
    rows = h * w
    kern = functools.partial(kern_fn, h=h, w=w)
    in_specs = list(x_specs) + [pl.BlockSpec(wt.shape, lambda c, i: (0, 0, 0))]
    args = list(xs) + [wt]
    if ss is not None:
        in_specs.append(pl.BlockSpec((2, co), lambda c, i: (0, 0)))
        args.append(ss)
    y, stats = pl.pallas_call(
        kern,
        out_shape=(jax.ShapeDtypeStruct((n * rows, co), jnp.bfloat16),
                   jax.ShapeDtypeStruct((2, ncores * co), jnp.float32)),
        grid=(ncores, per_core),
        in_specs=in_specs,
        out_specs=(
            pl.BlockSpec((rows, co), lambda c, i: (c * per_core + i, 0)),
            pl.BlockSpec((2, co), lambda c, i: (0, c)),
        ),
        compiler_params=pltpu.CompilerParams(
            dimension_semantics=("parallel", "arbitrary"),
            vmem_limit_bytes=VMEM_LIMIT_BYTES),
    )(*args)
    return y, stats.reshape(2, ncores, co).sum(axis=1)


def kernel(x, w1, w2, g1, b1, g2, b2):
    n, cin, hh, ww = x.shape
    h, w = hh // 2, ww // 2
    cmid, cout = w1.shape[0], w2.shape[0]
    rows = h * w
    cnt = jnp.float32(n * rows)
    ncores = 2 if n % 2 == 0 else 1
    per_core = n // ncores

    x4 = x.reshape(n * cin, h, 2, 1, ww)
    spec_e = pl.BlockSpec((cin, h, None, None, ww),
                          lambda c, i: (c * per_core + i, 0, 0, 0, 0))
    spec_o = pl.BlockSpec((cin, h, None, None, ww),
                          lambda c, i: (c * per_core + i, 0, 1, 0, 0))
    y1, stats1 = _conv_stats([x4, x4], [spec_e, spec_o], _wt(w1), None,
                             _conv1_kernel, h, w, n, cmid, ncores)
    ss1 = _fold_bn(stats1, g1, b1, cnt)

    xb_spec = pl.BlockSpec((rows, cmid), lambda c, i: (c * per_core + i, 0))
    y2, stats2 = _conv_stats([y1], [xb_spec], _wt(w2), ss1, _conv2_kernel,
                             h, w, n, cout, ncores)
    ss2 = _fold_bn(stats2, g2, b2, cnt)

    out = pl.pallas_call(
        _bn_out_kernel,
        out_shape=jax.ShapeDtypeStruct((n * cout, rows), jnp.float32),
        grid=(ncores, per_core),
        in_specs=[
            pl.BlockSpec((rows, cout), lambda c, i: (c * per_core + i, 0)),
            pl.BlockSpec((2, cout), lambda c, i: (0, 0)),
        ],
        out_specs=pl.BlockSpec((cout, rows), lambda c, i: (c * per_core + i, 0)),
        compiler_params=pltpu.CompilerParams(
            dimension_semantics=("parallel", "arbitrary"),
            vmem_limit_bytes=VMEM_LIMIT_BYTES),
    )(y2, ss2)
    return out.reshape(n, cout, h, w)


# speedup vs baseline: 13.3590x; 1.4002x over previous
import functools

import jax
import jax.numpy as jnp
from jax import lax
from jax.experimental import pallas as pl
from jax.experimental.pallas import tpu as pltpu

BN_EPS = 1e-5
VMEM_LIMIT_BYTES = 48 * 1024 * 1024


def _conv_body(v, w_ref, h, w):
    c = v.shape[-1]
    vm1 = jnp.concatenate([v[:, -1:], v[:, :-1]], axis=1)
    vp1 = jnp.concatenate([v[:, 1:], v[:, :1]], axis=1)
    xc = jnp.concatenate([vm1, v, vp1], axis=2)
    xcp = jnp.concatenate([xc[-1:], xc, xc[:1]], axis=0)
    xb = xcp.reshape((h + 2) * w, 3 * c)
    rows = h * w
    acc = jnp.dot(xb[0:rows], w_ref[0], preferred_element_type=jnp.float32)
    acc = acc + jnp.dot(xb[w:w + rows], w_ref[1],
                        preferred_element_type=jnp.float32)
    acc = acc + jnp.dot(xb[2 * w:2 * w + rows], w_ref[2],
                        preferred_element_type=jnp.float32)
    return acc


def _accum_stats(stats_ref, acc):
    ts = jnp.sum(acc, axis=0, keepdims=True)
    tq = jnp.sum(acc * acc, axis=0, keepdims=True)
    tot = jnp.concatenate([ts, tq], axis=0)

    @pl.when(pl.program_id(1) == 0)
    def _init():
        stats_ref[...] = jnp.zeros_like(stats_ref)

    stats_ref[...] += tot


def _conv1_kernel(x_ref, w_ref, y_ref, stats_ref, *, h, w):
    c = x_ref.shape[0]
    v = x_ref[...].reshape(c * 2 * h, 2 * w)
    ms = jnp.maximum(v, jnp.concatenate([v[:, 1:], v[:, :1]], axis=1))
    sel = (lax.broadcasted_iota(jnp.int32, (2 * w, w), 0)
           == 2 * lax.broadcasted_iota(jnp.int32, (2 * w, w), 1)
           ).astype(jnp.bfloat16)
    wp = jnp.dot(ms.astype(jnp.bfloat16), sel,
                 preferred_element_type=jnp.float32)
    t = jnp.transpose(wp.astype(jnp.bfloat16).reshape(c, 2 * h, w),
                      (1, 2, 0))
    vt = jnp.maximum(t.reshape(h, 2, w, c)[:, 0], t.reshape(h, 2, w, c)[:, 1])
    acc = _conv_body(vt, w_ref, h, w)
    y_ref[...] = acc.astype(jnp.bfloat16)
    _accum_stats(stats_ref, acc)


def _conv2_kernel(y1_ref, w_ref, ss_ref, y2_ref, stats_ref, *, h, w):
    a = jnp.maximum(y1_ref[...].astype(jnp.float32) * ss_ref[0] + ss_ref[1], 0.0)
    v = a.astype(jnp.bfloat16).reshape(h, w, a.shape[-1])
    acc = _conv_body(v, w_ref, h, w)
    y2_ref[...] = acc.astype(jnp.bfloat16)
    _accum_stats(stats_ref, acc)


def _bn_out_kernel(y2_ref, ss_ref, o_ref, *, h, w):
    a = jnp.maximum(y2_ref[...].astype(jnp.float32) * ss_ref[0] + ss_ref[1], 0.0)
    o_ref[...] = jnp.transpose(a.reshape(h, w, a.shape[-1]), (2, 0, 1))[None]


def _fold_bn(stats, gamma, beta, count):
    mean = stats[0] / count
    var = jnp.maximum(stats[1] / count - mean * mean, 0.0)
    inv = lax.rsqrt(var + BN_EPS)
    scale = gamma.astype(jnp.float32) * inv
    shift = beta.astype(jnp.float32) - mean * scale
    return jnp.stack([scale, shift], axis=0)


def _wt(weight):
    co, ci = weight.shape[0], weight.shape[1]
    return jnp.transpose(weight, (2, 3, 1, 0)).reshape(3, 3 * ci, co).astype(
        jnp.bfloat16)


def _conv_stats(xs, x_specs, wt, ss, kern_fn, h, w, n, co, ncores):
    per_core = n // ncores
    rows = h * w
    kern = functools.partial(kern_fn, h=h, w=w)
    in_specs = list(x_specs) + [pl.BlockSpec(wt.shape, lambda c, i: (0, 0, 0))]
    args = list(xs) + [wt]
    if ss is not None:
        in_specs.append(pl.BlockSpec((2, co), lambda c, i: (0, 0)))
        args.append(ss)
    y, stats = pl.pallas_call(
        kern,
        out_shape=(jax.ShapeDtypeStruct((n * rows, co), jnp.bfloat16),
                   jax.ShapeDtypeStruct((2, ncores * co), jnp.float32)),
        grid=(ncores, per_core),
        in_specs=in_specs,
        out_specs=(
            pl.BlockSpec((rows, co), lambda c, i: (c * per_core + i, 0)),
            pl.BlockSpec((2, co), lambda c, i: (0, c)),
        ),
        compiler_params=pltpu.CompilerParams(
            dimension_semantics=("parallel", "arbitrary"),
            vmem_limit_bytes=VMEM_LIMIT_BYTES),
    )(*args)
    return y, stats.reshape(2, ncores, co).sum(axis=1)


def kernel(x, w1, w2, g1, b1, g2, b2):
    n, cin, hh, ww = x.shape
    h, w = hh // 2, ww // 2
    cmid, cout = w1.shape[0], w2.shape[0]
    rows = h * w
    cnt = jnp.float32(n * rows)
    ncores = 2 if n % 2 == 0 else 1
    per_core = n // ncores

    x3 = x.reshape(n * cin, hh, ww)
    xa_spec = pl.BlockSpec((cin, hh, ww), lambda c, i: (c * per_core + i, 0, 0))
    y1, stats1 = _conv_stats([x3], [xa_spec], _wt(w1), None,
                             _conv1_kernel, h, w, n, cmid, ncores)
    ss1 = _fold_bn(stats1, g1, b1, cnt)

    xb_spec = pl.BlockSpec((rows, cmid), lambda c, i: (c * per_core + i, 0))
    y2, stats2 = _conv_stats([y1], [xb_spec], _wt(w2), ss1, _conv2_kernel,
                             h, w, n, cout, ncores)
    ss2 = _fold_bn(stats2, g2, b2, cnt)

    out = pl.pallas_call(
        functools.partial(_bn_out_kernel, h=h, w=w),
        out_shape=jax.ShapeDtypeStruct((n, cout, h, w), jnp.float32),
        grid=(ncores, per_core),
        in_specs=[
            pl.BlockSpec((rows, cout), lambda c, i: (c * per_core + i, 0)),
            pl.BlockSpec((2, cout), lambda c, i: (0, 0)),
        ],
        out_specs=pl.BlockSpec((1, cout, h, w),
                               lambda c, i: (c * per_core + i, 0, 0, 0)),
        compiler_params=pltpu.CompilerParams(
            dimension_semantics=("parallel", "arbitrary"),
            vmem_limit_bytes=VMEM_LIMIT_BYTES),
    )(y2, ss2)
    return out


# speedup vs baseline: 15.7960x; 1.1824x over previous
import functools

import jax
import jax.numpy as jnp
from jax import lax
from jax.experimental import pallas as pl
from jax.experimental.pallas import tpu as pltpu

BN_EPS = 1e-5
VMEM_LIMIT_BYTES = 48 * 1024 * 1024


def _conv_body(v, w_ref, h, w):
    c = v.shape[-1]
    vm1 = jnp.concatenate([v[:, -1:], v[:, :-1]], axis=1)
    vp1 = jnp.concatenate([v[:, 1:], v[:, :1]], axis=1)
    xc = jnp.concatenate([vm1, v, vp1], axis=2)
    xcp = jnp.concatenate([xc[-1:], xc, xc[:1]], axis=0)
    xb = xcp.reshape((h + 2) * w, 3 * c)
    rows = h * w
    acc = jnp.dot(xb[0:rows], w_ref[0], preferred_element_type=jnp.float32)
    acc = acc + jnp.dot(xb[w:w + rows], w_ref[1],
                        preferred_element_type=jnp.float32)
    acc = acc + jnp.dot(xb[2 * w:2 * w + rows], w_ref[2],
                        preferred_element_type=jnp.float32)
    return acc


def _accum_stats(stats_ref, acc):
    ts = jnp.sum(acc, axis=0, keepdims=True)
    tq = jnp.sum(acc * acc, axis=0, keepdims=True)
    tot = jnp.concatenate([ts, tq], axis=0)

    @pl.when(pl.program_id(1) == 0)
    def _init():
        stats_ref[...] = jnp.zeros_like(stats_ref)

    stats_ref[...] += tot


def _conv1_kernel(x_ref, w_ref, y_ref, stats_ref, *, h, w):
    c = x_ref.shape[0]
    v = x_ref[...].reshape(c * 2 * h, 2 * w)
    ms = jnp.maximum(v, jnp.concatenate([v[:, 1:], v[:, :1]], axis=1))
    sel = (lax.broadcasted_iota(jnp.int32, (2 * w, w), 0)
           == 2 * lax.broadcasted_iota(jnp.int32, (2 * w, w), 1)
           ).astype(jnp.bfloat16)
    wp = jnp.dot(ms.astype(jnp.bfloat16), sel,
                 preferred_element_type=jnp.float32)
    t = jnp.transpose(wp.astype(jnp.bfloat16).reshape(c, 2 * h, w),
                      (1, 2, 0))
    vt = jnp.maximum(t.reshape(h, 2, w, c)[:, 0], t.reshape(h, 2, w, c)[:, 1])
    acc = _conv_body(vt, w_ref, h, w)
    y_ref[...] = acc.astype(jnp.bfloat16)
    _accum_stats(stats_ref, acc)


def _conv2_kernel(y1_ref, w_ref, ss_ref, y2_ref, stats_ref, *, h, w):
    a = jnp.maximum(y1_ref[...].astype(jnp.float32) * ss_ref[0] + ss_ref[1], 0.0)
    v = a.astype(jnp.bfloat16).reshape(h, w, a.shape[-1])
    acc = _conv_body(v, w_ref, h, w)
    y2_ref[...] = acc.astype(jnp.bfloat16)
    _accum_stats(stats_ref, acc)


def _bn_out_kernel(y2_ref, ss_ref, o_ref):
    a = jnp.maximum(y2_ref[...].astype(jnp.float32) * ss_ref[0] + ss_ref[1], 0.0)
    o_ref[...] = a.T[None]


def _fold_bn(stats, gamma, beta, count):
    mean = stats[0] / count
    var = jnp.maximum(stats[1] / count - mean * mean, 0.0)
    inv = lax.rsqrt(var + BN_EPS)
    scale = gamma.astype(jnp.float32) * inv
    shift = beta.astype(jnp.float32) - mean * scale
    return jnp.stack([scale, shift], axis=0)


def _wt(weight):
    co, ci = weight.shape[0], weight.shape[1]
    return jnp.transpose(weight, (2, 3, 1, 0)).reshape(3, 3 * ci, co).astype(
        jnp.bfloat16)


def _conv_stats(xs, x_specs, wt, ss, kern_fn, h, w, n, co, ncores):
    per_core = n // ncores
    rows = h * w
    kern = functools.partial(kern_fn, h=h, w=w)
    in_specs = list(x_specs) + [pl.BlockSpec(wt.shape, lambda c, i: (0, 0, 0))]
    args = list(xs) + [wt]
    if ss is not None:
        in_specs.append(pl.BlockSpec((2, co), lambda c, i: (0, 0)))
        args.append(ss)
    y, stats = pl.pallas_call(
        kern,
        out_shape=(jax.ShapeDtypeStruct((n * rows, co), jnp.bfloat16),
                   jax.ShapeDtypeStruct((2, ncores * co), jnp.float32)),
        grid=(ncores, per_core),
        in_specs=in_specs,
        out_specs=(
            pl.BlockSpec((rows, co), lambda c, i: (c * per_core + i, 0)),
            pl.BlockSpec((2, co), lambda c, i: (0, c)),
        ),
        compiler_params=pltpu.CompilerParams(
            dimension_semantics=("parallel", "arbitrary"),
            vmem_limit_bytes=VMEM_LIMIT_BYTES),
    )(*args)
    return y, stats.reshape(2, ncores, co).sum(axis=1)


def kernel(x, w1, w2, g1, b1, g2, b2):
    n, cin, hh, ww = x.shape
    h, w = hh // 2, ww // 2
    cmid, cout = w1.shape[0], w2.shape[0]
    rows = h * w
    cnt = jnp.float32(n * rows)
    ncores = 2 if n % 2 == 0 else 1
    per_core = n // ncores

    x3 = x.reshape(n * cin, hh, ww)
    xa_spec = pl.BlockSpec((cin, hh, ww), lambda c, i: (c * per_core + i, 0, 0))
    y1, stats1 = _conv_stats([x3], [xa_spec], _wt(w1), None,
                             _conv1_kernel, h, w, n, cmid, ncores)
    ss1 = _fold_bn(stats1, g1, b1, cnt)

    xb_spec = pl.BlockSpec((rows, cmid), lambda c, i: (c * per_core + i, 0))
    y2, stats2 = _conv_stats([y1], [xb_spec], _wt(w2), ss1, _conv2_kernel,
                             h, w, n, cout, ncores)
    ss2 = _fold_bn(stats2, g2, b2, cnt)

    out = pl.pallas_call(
        _bn_out_kernel,
        out_shape=jax.ShapeDtypeStruct((n, cout, rows), jnp.float32),
        grid=(ncores, per_core),
        in_specs=[
            pl.BlockSpec((rows, cout), lambda c, i: (c * per_core + i, 0)),
            pl.BlockSpec((2, cout), lambda c, i: (0, 0)),
        ],
        out_specs=pl.BlockSpec((1, cout, rows),
                               lambda c, i: (c * per_core + i, 0, 0)),
        compiler_params=pltpu.CompilerParams(
            dimension_semantics=("parallel", "arbitrary"),
            vmem_limit_bytes=VMEM_LIMIT_BYTES),
    )(y2, ss2)
    return out.reshape(n, cout, h, w)


# speedup vs baseline: 17.7082x; 1.1211x over previous
import functools

import jax
import jax.numpy as jnp
from jax import lax
from jax.experimental import pallas as pl
from jax.experimental.pallas import tpu as pltpu

BN_EPS = 1e-5
VMEM_LIMIT_BYTES = 48 * 1024 * 1024


def _conv_body(v, w_ref, h, w):
    c = v.shape[-1]
    vm1 = jnp.concatenate([v[:, -1:], v[:, :-1]], axis=1)
    vp1 = jnp.concatenate([v[:, 1:], v[:, :1]], axis=1)
    xc = jnp.concatenate([vm1, v, vp1], axis=2)
    xcp = jnp.concatenate([xc[-1:], xc, xc[:1]], axis=0)
    xb = xcp.reshape((h + 2) * w, 3 * c)
    rows = h * w
    acc = jnp.dot(xb[0:rows], w_ref[0], preferred_element_type=jnp.float32)
    acc = acc + jnp.dot(xb[w:w + rows], w_ref[1],
                        preferred_element_type=jnp.float32)
    acc = acc + jnp.dot(xb[2 * w:2 * w + rows], w_ref[2],
                        preferred_element_type=jnp.float32)
    return acc


def _stat_of(acc):
    ts = jnp.sum(acc, axis=0, keepdims=True)
    tq = jnp.sum(acc * acc, axis=0, keepdims=True)
    return jnp.concatenate([ts, tq], axis=0)


def _accum_stats(stats_ref, tot):
    @pl.when(pl.program_id(1) == 0)
    def _init():
        stats_ref[...] = jnp.zeros_like(stats_ref)

    stats_ref[...] += tot


def _conv1_kernel(x_ref, w_ref, y_ref, stats_ref, *, h, w, k, cin):
    rows = h * w
    sel = (lax.broadcasted_iota(jnp.int32, (2 * w, w), 0)
           == 2 * lax.broadcasted_iota(jnp.int32, (2 * w, w), 1)
           ).astype(jnp.bfloat16)
    tot = jnp.zeros((2, y_ref.shape[-1]), jnp.float32)
    for j in range(k):
        v = x_ref[j * cin:(j + 1) * cin].reshape(cin * 2 * h, 2 * w)
        ms = jnp.maximum(v, jnp.concatenate([v[:, 1:], v[:, :1]], axis=1))
        wp = jnp.dot(ms.astype(jnp.bfloat16), sel,
                     preferred_element_type=jnp.float32)
        t = jnp.transpose(wp.astype(jnp.bfloat16).reshape(cin, 2 * h, w),
                          (1, 2, 0))
        vt = jnp.maximum(t.reshape(h, 2, w, cin)[:, 0],
                         t.reshape(h, 2, w, cin)[:, 1])
        acc = _conv_body(vt, w_ref, h, w)
        y_ref[j * rows:(j + 1) * rows, :] = acc.astype(jnp.bfloat16)
        tot = tot + _stat_of(acc)
    _accum_stats(stats_ref, tot)


def _conv2_kernel(y1_ref, w_ref, ss_ref, y2_ref, stats_ref, *, h, w, k):
    rows = h * w
    tot = jnp.zeros((2, y2_ref.shape[-1]), jnp.float32)
    for j in range(k):
        a = jnp.maximum(
            y1_ref[j * rows:(j + 1) * rows].astype(jnp.float32) * ss_ref[0]
            + ss_ref[1], 0.0)
        v = a.astype(jnp.bfloat16).reshape(h, w, a.shape[-1])
        acc = _conv_body(v, w_ref, h, w)
        y2_ref[j * rows:(j + 1) * rows, :] = acc.astype(jnp.bfloat16)
        tot = tot + _stat_of(acc)
    _accum_stats(stats_ref, tot)


def _bn_out_kernel(y2_ref, ss_ref, o_ref, *, k):
    rows = y2_ref.shape[0] // k
    for j in range(k):
        a = jnp.maximum(
            y2_ref[j * rows:(j + 1) * rows].astype(jnp.float32) * ss_ref[0]
            + ss_ref[1], 0.0)
        o_ref[j] = a.T


def _fold_bn(stats, gamma, beta, count):
    mean = stats[0] / count
    var = jnp.maximum(stats[1] / count - mean * mean, 0.0)
    inv = lax.rsqrt(var + BN_EPS)
    scale = gamma.astype(jnp.float32) * inv
    shift = beta.astype(jnp.float32) - mean * scale
    return jnp.stack([scale, shift], axis=0)


def _wt(weight):
    co, ci = weight.shape[0], weight.shape[1]
    return jnp.transpose(weight, (2, 3, 1, 0)).reshape(3, 3 * ci, co).astype(
        jnp.bfloat16)


def _conv_stats(xs, x_specs, wt, ss, kern, h, w, n, co, ncores, k):
    steps = n // ncores // k
    rows = h * w
    in_specs = list(x_specs) + [pl.BlockSpec(wt.shape, lambda c, i: (0, 0, 0))]
    args = list(xs) + [wt]
    if ss is not None:
        in_specs.append(pl.BlockSpec((2, co), lambda c, i: (0, 0)))
        args.append(ss)
    y, stats = pl.pallas_call(
        kern,
        out_shape=(jax.ShapeDtypeStruct((n * rows, co), jnp.bfloat16),
                   jax.ShapeDtypeStruct((2, ncores * co), jnp.float32)),
        grid=(ncores, steps),
        in_specs=in_specs,
        out_specs=(
            pl.BlockSpec((k * rows, co), lambda c, i: (c * steps + i, 0)),
            pl.BlockSpec((2, co), lambda c, i: (0, c)),
        ),
        compiler_params=pltpu.CompilerParams(
            dimension_semantics=("parallel", "arbitrary"),
            vmem_limit_bytes=VMEM_LIMIT_BYTES),
    )(*args)
    return y, stats.reshape(2, ncores, co).sum(axis=1)


def kernel(x, w1, w2, g1, b1, g2, b2):
    n, cin, hh, ww = x.shape
    h, w = hh // 2, ww // 2
    cmid, cout = w1.shape[0], w2.shape[0]
    rows = h * w
    cnt = jnp.float32(n * rows)
    ncores = 2 if n % 2 == 0 else 1
    per_core = n // ncores
    k = 2 if per_core % 2 == 0 else 1
    steps = per_core // k

    x3 = x.reshape(n * cin, hh, ww)
    xa_spec = pl.BlockSpec((k * cin, hh, ww),
                           lambda c, i: (c * steps + i, 0, 0))
    kern_a = functools.partial(_conv1_kernel, h=h, w=w, k=k, cin=cin)
    y1, stats1 = _conv_stats([x3], [xa_spec], _wt(w1), None,
                             kern_a, h, w, n, cmid, ncores, k)
    ss1 = _fold_bn(stats1, g1, b1, cnt)

    xb_spec = pl.BlockSpec((k * rows, cmid), lambda c, i: (c * steps + i, 0))
    kern_b = functools.partial(_conv2_kernel, h=h, w=w, k=k)
    y2, stats2 = _conv_stats([y1], [xb_spec], _wt(w2), ss1, kern_b,
                             h, w, n, cout, ncores, k)
    ss2 = _fold_bn(stats2, g2, b2, cnt)

    out = pl.pallas_call(
        functools.partial(_bn_out_kernel, k=k),
        out_shape=jax.ShapeDtypeStruct((n, cout, rows), jnp.float32),
        grid=(ncores, steps),
        in_specs=[
            pl.BlockSpec((k * rows, cout), lambda c, i: (c * steps + i, 0)),
            pl.BlockSpec((2, cout), lambda c, i: (0, 0)),
        ],
        out_specs=pl.BlockSpec((k, cout, rows),
                               lambda c, i: (c * steps + i, 0, 0)),
        compiler_params=pltpu.CompilerParams(
            dimension_semantics=("parallel", "arbitrary"),
            vmem_limit_bytes=VMEM_LIMIT_BYTES),
    )(y2, ss2)
    return out.reshape(n, cout, h, w)
